# baseline (speedup 1.0000x reference)
import functools

import jax
import jax.numpy as jnp
from jax import lax
from jax.experimental import pallas as pl
from jax.experimental.pallas import tpu as pltpu
from jax.experimental.pallas import tpu_sc as plsc

C = 1000
B = 16384
NC, NS, L = 2, 16, 16
NW = NC * NS
ROWS_PER_W = B // NW
CHUNK = 32
NCHUNK = ROWS_PER_W // CHUNK

_mesh = plsc.VectorSubcoreMesh(core_axis_name="c", subcore_axis_name="s")


@functools.partial(
    pl.kernel,
    mesh=_mesh,
    out_type=jax.ShapeDtypeStruct((B, C), jnp.float32),
    scratch_types=[
        pltpu.VMEM((ROWS_PER_W,), jnp.int32),
        pltpu.VMEM((CHUNK, C), jnp.float32),
        pltpu.VMEM((CHUNK, C), jnp.float32),
        pltpu.SemaphoreType.DMA,
        pltpu.SemaphoreType.DMA,
    ],
    compiler_params=pltpu.CompilerParams(
        needs_layout_passes=False, use_tc_tiling_on_sc=True
    ),
)
def _onehot_sc(y_hbm, out_hbm, idx_v, buf0, buf1, sem0, sem1):
    sid = lax.axis_index("s")
    wid = sid * NC + lax.axis_index("c")
    base = wid * ROWS_PER_W
    pltpu.sync_copy(y_hbm.at[pl.ds(base, ROWS_PER_W)], idx_v)

    zeros16 = jnp.zeros((L,), jnp.float32)
    ones16 = jnp.ones((L,), jnp.float32)

    def zero_body(i, carry):
        for r in range(CHUNK):
            buf0[r, pl.ds(i * L, L)] = zeros16
            buf1[r, pl.ds(i * L, L)] = zeros16
        return carry

    lax.fori_loop(0, C // L, zero_body, 0)
    for r in range(CHUNK):
        buf0[r, pl.ds(C - L, L)] = zeros16
        buf1[r, pl.ds(C - L, L)] = zeros16

    iota = lax.iota(jnp.int32, L)
    rows0 = iota
    rows1 = iota + L

    def positions(c):
        y0 = idx_v[pl.ds(c * CHUNK, L)]
        y1 = idx_v[pl.ds(c * CHUNK + L, L)]
        return y0, y1

    bufs = (buf0, buf1)
    sems = (sem0, sem1)
    copies = [None, None]
    for c in range(NCHUNK):
        bsel = c % 2
        buf, sem = bufs[bsel], sems[bsel]
        if c >= 2:
            copies[bsel].wait()
            q0, q1 = positions(c - 2)
            plsc.store_scatter(buf, [rows0, q0], zeros16)
            plsc.store_scatter(buf, [rows1, q1], zeros16)
        p0, p1 = positions(c)
        plsc.store_scatter(buf, [rows0, p0], ones16)
        plsc.store_scatter(buf, [rows1, p1], ones16)
        dst = out_hbm.at[pl.ds(base + c * CHUNK, CHUNK), :]
        copies[bsel] = pltpu.async_copy(buf, dst, sem)
    copies[0].wait()
    copies[1].wait()


def kernel(y_n):
    return _onehot_sc(y_n)


# speedup vs baseline: 1.8522x; 1.8522x over previous
import functools

import jax
import jax.numpy as jnp
from jax import lax
from jax.experimental import pallas as pl
from jax.experimental.pallas import tpu as pltpu
from jax.experimental.pallas import tpu_sc as plsc

C = 1000
B = 16384
NC, NS, L = 2, 16, 16
NW = NC * NS
BW = B // NW
CBLK = 64
NCHUNK = C // CBLK
CTAIL = C - NCHUNK * CBLK

_mesh = plsc.VectorSubcoreMesh(core_axis_name="c", subcore_axis_name="s")


@functools.partial(
    pl.kernel,
    mesh=_mesh,
    out_type=jax.ShapeDtypeStruct((C, B), jnp.float32),
    scratch_types=[
        pltpu.VMEM((BW,), jnp.int32),
        pltpu.VMEM((CBLK, BW), jnp.float32),
        pltpu.VMEM((CBLK, BW), jnp.float32),
        pltpu.VMEM((CTAIL, BW), jnp.float32),
        pltpu.SemaphoreType.DMA,
        pltpu.SemaphoreType.DMA,
        pltpu.SemaphoreType.DMA,
    ],
    compiler_params=pltpu.CompilerParams(needs_layout_passes=False),
)
def _onehot_sc(y_hbm, out_hbm, idx_v, buf0, buf1, buft, sem0, sem1, semt):
    sid = lax.axis_index("s")
    wid = sid * NC + lax.axis_index("c")
    base = wid * BW
    pltpu.sync_copy(y_hbm.at[pl.ds(base, BW)], idx_v)

    zeros16 = jnp.zeros((L,), jnp.float32)
    ones16 = jnp.ones((L,), jnp.float32)
    iota = lax.iota(jnp.int32, L)

    def zero_body(i, carry):
        for r in range(CBLK):
            buf0[r, pl.ds(i * L, L)] = zeros16
            buf1[r, pl.ds(i * L, L)] = zeros16
        for r in range(CTAIL):
            buft[r, pl.ds(i * L, L)] = zeros16
        return carry

    lax.fori_loop(0, BW // L, zero_body, 0)

    def scatter(buf, c0, width, val16):
        for k in range(BW // L):
            y16 = idx_v[pl.ds(k * L, L)]
            row = y16 - c0
            col = iota + (k * L)
            mask = (y16 >= c0) & (y16 < c0 + width)
            plsc.store_scatter(buf, [row, col], val16, mask=mask)

    bufs = (buf0, buf1)
    sems = (sem0, sem1)
    copies = [None, None]
    for cc in range(NCHUNK):
        bsel = cc % 2
        buf, sem = bufs[bsel], sems[bsel]
        if cc >= 2:
            copies[bsel].wait()
            scatter(buf, (cc - 2) * CBLK, CBLK, zeros16)
        scatter(buf, cc * CBLK, CBLK, ones16)
        dst = out_hbm.at[pl.ds(cc * CBLK, CBLK), pl.ds(base, BW)]
        copies[bsel] = pltpu.async_copy(buf, dst, sem)
    scatter(buft, NCHUNK * CBLK, CTAIL, ones16)
    dstt = out_hbm.at[pl.ds(NCHUNK * CBLK, CTAIL), pl.ds(base, BW)]
    tcopy = pltpu.async_copy(buft, dstt, semt)
    copies[0].wait()
    copies[1].wait()
    tcopy.wait()


def kernel(y_n):
    return _onehot_sc(y_n).T


# speedup vs baseline: 2.2370x; 1.2077x over previous
import functools

import jax
import jax.numpy as jnp
from jax import lax
from jax.experimental import pallas as pl
from jax.experimental.pallas import tpu as pltpu
from jax.experimental.pallas import tpu_sc as plsc

C = 1000
B = 16384
NC, NS, L = 2, 16, 16
NW = NC * NS
BW = B // NW
CBLK = 64
NCHUNK = C // CBLK
CTAIL = C - NCHUNK * CBLK

_mesh = plsc.VectorSubcoreMesh(core_axis_name="c", subcore_axis_name="s")


@functools.partial(
    pl.kernel,
    mesh=_mesh,
    out_type=jax.ShapeDtypeStruct((C, B), jnp.float32),
    scratch_types=[
        pltpu.VMEM((BW,), jnp.int32),
        pltpu.VMEM((CBLK, BW), jnp.float32),
        pltpu.VMEM((CBLK, BW), jnp.float32),
        pltpu.VMEM((CTAIL, BW), jnp.float32),
        pltpu.SemaphoreType.DMA,
        pltpu.SemaphoreType.DMA,
        pltpu.SemaphoreType.DMA,
    ],
    compiler_params=pltpu.CompilerParams(needs_layout_passes=False),
)
def _onehot_sc(y_hbm, out_hbm, idx_v, buf0, buf1, buft, sem0, sem1, semt):
    sid = lax.axis_index("s")
    wid = sid * NC + lax.axis_index("c")
    base = wid * BW
    pltpu.sync_copy(y_hbm.at[pl.ds(base, BW)], idx_v)

    zeros16 = jnp.zeros((L,), jnp.float32)
    ones16 = jnp.ones((L,), jnp.float32)
    iota = lax.iota(jnp.int32, L)

    def zero_body(i, carry):
        for r in range(CBLK):
            buf0[r, pl.ds(i * L, L)] = zeros16
            buf1[r, pl.ds(i * L, L)] = zeros16
        for r in range(CTAIL):
            buft[r, pl.ds(i * L, L)] = zeros16
        return carry

    lax.fori_loop(0, BW // L, zero_body, 0)

    def scatter(buf, c0, width, val16):
        def body(k, carry):
            y16 = idx_v[pl.ds(k * L, L)]
            row = y16 - c0
            col = iota + k * L
            mask = (y16 >= c0) & (y16 < c0 + width)
            plsc.store_scatter(buf, [row, col], val16, mask=mask)
            return carry

        lax.fori_loop(0, BW // L, body, 0)

    bufs = (buf0, buf1)
    sems = (sem0, sem1)
    copies = [None, None]
    for cc in range(NCHUNK):
        bsel = cc % 2
        buf, sem = bufs[bsel], sems[bsel]
        if cc >= 2:
            copies[bsel].wait()
            scatter(buf, (cc - 2) * CBLK, CBLK, zeros16)
        scatter(buf, cc * CBLK, CBLK, ones16)
        dst = out_hbm.at[pl.ds(cc * CBLK, CBLK), pl.ds(base, BW)]
        copies[bsel] = pltpu.async_copy(buf, dst, sem)
    scatter(buft, NCHUNK * CBLK, CTAIL, ones16)
    dstt = out_hbm.at[pl.ds(NCHUNK * CBLK, CTAIL), pl.ds(base, BW)]
    tcopy = pltpu.async_copy(buft, dstt, semt)
    copies[0].wait()
    copies[1].wait()
    tcopy.wait()


def kernel(y_n):
    return _onehot_sc(y_n).T


# speedup vs baseline: 2.3057x; 1.0307x over previous
import functools

import jax
import jax.numpy as jnp
from jax import lax
from jax.experimental import pallas as pl
from jax.experimental.pallas import tpu as pltpu
from jax.experimental.pallas import tpu_sc as plsc

C = 1000
B = 16384
NC, NS, L = 2, 16, 16
NW = NC * NS
BW = B // NW
CBLK = 64
NCHUNK = C // CBLK
CTAIL = C - NCHUNK * CBLK
NBUF = 3

_mesh = plsc.VectorSubcoreMesh(core_axis_name="c", subcore_axis_name="s")


@functools.partial(
    pl.kernel,
    mesh=_mesh,
    out_type=jax.ShapeDtypeStruct((C, B), jnp.float32),
    scratch_types=[
        pltpu.VMEM((BW,), jnp.int32),
        pltpu.VMEM((CBLK, BW), jnp.float32),
        pltpu.VMEM((CBLK, BW), jnp.float32),
        pltpu.VMEM((CBLK, BW), jnp.float32),
        pltpu.VMEM((CTAIL, BW), jnp.float32),
        pltpu.SemaphoreType.DMA,
        pltpu.SemaphoreType.DMA,
        pltpu.SemaphoreType.DMA,
        pltpu.SemaphoreType.DMA,
    ],
    compiler_params=pltpu.CompilerParams(needs_layout_passes=False),
)
def _onehot_sc(y_hbm, out_hbm, idx_v, buf0, buf1, buf2, buft,
               sem0, sem1, sem2, semt):
    sid = lax.axis_index("s")
    wid = sid * NC + lax.axis_index("c")
    base = wid * BW
    pltpu.sync_copy(y_hbm.at[pl.ds(base, BW)], idx_v)

    zeros16 = jnp.zeros((L,), jnp.float32)
    ones16 = jnp.ones((L,), jnp.float32)
    iota = lax.iota(jnp.int32, L)

    def zero_buf(buf, rows):
        def body(i, carry):
            for r in range(rows):
                buf[r, pl.ds(i * L, L)] = zeros16
            return carry

        lax.fori_loop(0, BW // L, body, 0)

    def scatter(buf, c0, width, val16):
        def body(k, carry):
            y16 = idx_v[pl.ds(k * L, L)]
            row = y16 - c0
            col = iota + k * L
            mask = (y16 >= c0) & (y16 < c0 + width)
            plsc.store_scatter(buf, [row, col], val16, mask=mask)
            return carry

        lax.fori_loop(0, BW // L, body, 0)

    bufs = (buf0, buf1, buf2)
    sems = (sem0, sem1, sem2)
    copies = [None, None, None]
    for cc in range(NCHUNK):
        bsel = cc % NBUF
        buf, sem = bufs[bsel], sems[bsel]
        if cc < NBUF:
            zero_buf(buf, CBLK)
        else:
            copies[bsel].wait()
            scatter(buf, (cc - NBUF) * CBLK, CBLK, zeros16)
        scatter(buf, cc * CBLK, CBLK, ones16)
        dst = out_hbm.at[pl.ds(cc * CBLK, CBLK), pl.ds(base, BW)]
        copies[bsel] = pltpu.async_copy(buf, dst, sem)
    zero_buf(buft, CTAIL)
    scatter(buft, NCHUNK * CBLK, CTAIL, ones16)
    dstt = out_hbm.at[pl.ds(NCHUNK * CBLK, CTAIL), pl.ds(base, BW)]
    tcopy = pltpu.async_copy(buft, dstt, semt)
    for cp in copies:
        cp.wait()
    tcopy.wait()


def kernel(y_n):
    return _onehot_sc(y_n).T


# speedup vs baseline: 2.3138x; 1.0035x over previous
import functools

import jax
import jax.numpy as jnp
from jax import lax
from jax.experimental import pallas as pl
from jax.experimental.pallas import tpu as pltpu
from jax.experimental.pallas import tpu_sc as plsc

C = 1000
B = 16384
NC, NS, L = 2, 16, 16
NW = NC * NS
BW = B // NW
CBLK = 64
NCHUNK = C // CBLK
CTAIL = C - NCHUNK * CBLK
NBUF = 3

_mesh = plsc.VectorSubcoreMesh(core_axis_name="c", subcore_axis_name="s")


@functools.partial(
    pl.kernel,
    mesh=_mesh,
    out_type=jax.ShapeDtypeStruct((C, B), jnp.float32),
    scratch_types=[
        pltpu.VMEM((BW,), jnp.int32),
        pltpu.VMEM((CBLK, BW), jnp.float32),
        pltpu.VMEM((CBLK, BW), jnp.float32),
        pltpu.VMEM((CBLK, BW), jnp.float32),
        pltpu.VMEM((CTAIL, BW), jnp.float32),
        pltpu.SemaphoreType.DMA,
        pltpu.SemaphoreType.DMA,
        pltpu.SemaphoreType.DMA,
        pltpu.SemaphoreType.DMA,
    ],
    compiler_params=pltpu.CompilerParams(
        needs_layout_passes=False, skip_device_barrier=True
    ),
)
def _onehot_sc(y_hbm, out_hbm, idx_v, buf0, buf1, buf2, buft,
               sem0, sem1, sem2, semt):
    sid = lax.axis_index("s")
    wid = sid * NC + lax.axis_index("c")
    base = wid * BW
    pltpu.sync_copy(y_hbm.at[pl.ds(base, BW)], idx_v)

    zeros16 = jnp.zeros((L,), jnp.float32)
    ones16 = jnp.ones((L,), jnp.float32)
    iota = lax.iota(jnp.int32, L)

    def zero_buf(buf, rows):
        def body(i, carry):
            for r in range(rows):
                buf[r, pl.ds(i * L, L)] = zeros16
            return carry

        lax.fori_loop(0, BW // L, body, 0)

    def scatter(buf, c0, width, val16):
        def body(k, carry):
            y16 = idx_v[pl.ds(k * L, L)]
            row = y16 - c0
            col = iota + k * L
            mask = (y16 >= c0) & (y16 < c0 + width)
            plsc.store_scatter(buf, [row, col], val16, mask=mask)
            return carry

        lax.fori_loop(0, BW // L, body, 0)

    bufs = (buf0, buf1, buf2)
    sems = (sem0, sem1, sem2)
    copies = [None, None, None]
    for cc in range(NCHUNK):
        bsel = cc % NBUF
        buf, sem = bufs[bsel], sems[bsel]
        if cc < NBUF:
            zero_buf(buf, CBLK)
        else:
            copies[bsel].wait()
            scatter(buf, (cc - NBUF) * CBLK, CBLK, zeros16)
        scatter(buf, cc * CBLK, CBLK, ones16)
        dst = out_hbm.at[pl.ds(cc * CBLK, CBLK), pl.ds(base, BW)]
        copies[bsel] = pltpu.async_copy(buf, dst, sem)
    zero_buf(buft, CTAIL)
    scatter(buft, NCHUNK * CBLK, CTAIL, ones16)
    dstt = out_hbm.at[pl.ds(NCHUNK * CBLK, CTAIL), pl.ds(base, BW)]
    tcopy = pltpu.async_copy(buft, dstt, semt)
    for cp in copies:
        cp.wait()
    tcopy.wait()


def kernel(y_n):
    return _onehot_sc(y_n).T


# speedup vs baseline: 2.3141x; 1.0001x over previous
import functools

import jax
import jax.numpy as jnp
from jax import lax
from jax.experimental import pallas as pl
from jax.experimental.pallas import tpu as pltpu
from jax.experimental.pallas import tpu_sc as plsc

C = 1000
B = 16384
NC, NS, L = 2, 16, 16
NW = NC * NS
BW = B // NW
CBLK = 64
NCHUNK = C // CBLK
CTAIL = C - NCHUNK * CBLK
NBUF = 3

_mesh = plsc.VectorSubcoreMesh(core_axis_name="c", subcore_axis_name="s")


@functools.partial(
    pl.kernel,
    mesh=_mesh,
    out_type=jax.ShapeDtypeStruct((C, B), jnp.float32),
    scratch_types=[
        pltpu.VMEM((BW,), jnp.int32),
        pltpu.VMEM((CBLK, BW), jnp.float32),
        pltpu.VMEM((CBLK, BW), jnp.float32),
        pltpu.VMEM((CBLK, BW), jnp.float32),
        pltpu.VMEM((CTAIL, BW), jnp.float32),
        pltpu.SemaphoreType.DMA,
        pltpu.SemaphoreType.DMA,
        pltpu.SemaphoreType.DMA,
        pltpu.SemaphoreType.DMA,
    ],
    compiler_params=pltpu.CompilerParams(needs_layout_passes=False),
)
def _onehot_sc(y_hbm, out_hbm, idx_v, buf0, buf1, buf2, buft,
               sem0, sem1, sem2, semt):
    sid = lax.axis_index("s")
    wid = sid * NC + lax.axis_index("c")
    base = wid * BW
    pltpu.sync_copy(y_hbm.at[pl.ds(base, BW)], idx_v)

    zeros16 = jnp.zeros((L,), jnp.float32)
    ones16 = jnp.ones((L,), jnp.float32)
    iota = lax.iota(jnp.int32, L)

    def zero_buf(buf, rows):
        def body(i, carry):
            for r in range(rows):
                buf[r, pl.ds(i * L, L)] = zeros16
            return carry

        lax.fori_loop(0, BW // L, body, 0)

    def scatter(buf, c0, width, val16):
        def body(k, carry):
            y16 = idx_v[pl.ds(k * L, L)]
            row = y16 - c0
            col = iota + k * L
            mask = (y16 >= c0) & (y16 < c0 + width)
            plsc.store_scatter(buf, [row, col], val16, mask=mask)
            return carry

        lax.fori_loop(0, BW // L, body, 0)

    bufs = (buf0, buf1, buf2)
    sems = (sem0, sem1, sem2)
    copies = [None, None, None]
    for cc in range(NCHUNK):
        bsel = cc % NBUF
        buf, sem = bufs[bsel], sems[bsel]
        if cc < NBUF:
            zero_buf(buf, CBLK)
        else:
            copies[bsel].wait()
            scatter(buf, (cc - NBUF) * CBLK, CBLK, zeros16)
        scatter(buf, cc * CBLK, CBLK, ones16)
        dst = out_hbm.at[pl.ds(cc * CBLK, CBLK), pl.ds(base, BW)]
        copies[bsel] = pltpu.async_copy(buf, dst, sem)
    zero_buf(buft, CTAIL)
    scatter(buft, NCHUNK * CBLK, CTAIL, ones16)
    dstt = out_hbm.at[pl.ds(NCHUNK * CBLK, CTAIL), pl.ds(base, BW)]
    tcopy = pltpu.async_copy(buft, dstt, semt)
    for cp in copies:
        cp.wait()
    tcopy.wait()


def kernel(y_n):
    return _onehot_sc(y_n).T
